# Initial kernel scaffold; baseline (speedup 1.0000x reference)
import jax
import jax.numpy as jnp
from jax.experimental import pallas as pl


def kernel(x, Wg, bg, W1, B1, W2, B2, W3, B3):
    raise NotImplementedError("write your pallas kernel here")


# speedup vs baseline: 3.0737x; 3.0737x over previous
import functools

import jax
import jax.numpy as jnp
from jax.experimental import pallas as pl
from jax.experimental.pallas import tpu as pltpu

TILE = 64
TOPK = 2


def _gating_body(xf_ref, wg_ref, bg_ref, idx_ref, w_ref):
    xf = xf_ref[...]
    logits = jax.lax.dot_general(
        xf, wg_ref[...], (((1,), (1,)), ((), ())),
        preferred_element_type=jnp.float32) + bg_ref[...]
    n, e = logits.shape
    ids = jax.lax.broadcasted_iota(jnp.int32, (n, e), 1)
    m1 = jnp.max(logits, axis=1, keepdims=True)
    top1 = jnp.min(jnp.where(logits == m1, ids, e), axis=1)
    masked = jnp.where(ids == top1[:, None], -jnp.inf, logits)
    m2 = jnp.max(masked, axis=1, keepdims=True)
    top2 = jnp.min(jnp.where(masked == m2, ids, e), axis=1)
    w2 = jax.nn.sigmoid(m2[:, 0] - m1[:, 0])
    idx_ref[...] = jnp.stack([top1, top2], axis=1)
    w_ref[...] = jnp.stack([1.0 - w2, w2], axis=1)


def _gating(xf, Wg, bg):
    n, d = xf.shape
    e = Wg.shape[0]
    return pl.pallas_call(
        _gating_body,
        out_shape=(jax.ShapeDtypeStruct((n, TOPK), jnp.int32),
                   jax.ShapeDtypeStruct((n, TOPK), jnp.float32)),
    )(xf, Wg, bg.reshape(1, e))


def _route(idx, n_experts):
    n, k = idx.shape
    a = n * k
    nt = a // TILE + n_experts
    np_ = nt * TILE
    eids = idx.reshape(a)
    order = jnp.argsort(eids, stable=True).astype(jnp.int32)
    es = eids[order]
    counts = jnp.zeros((n_experts,), jnp.int32).at[eids].add(1)
    csum = jnp.cumsum(counts)
    off = csum - counts
    padded = ((counts + TILE - 1) // TILE) * TILE
    pcsum = jnp.cumsum(padded)
    poff = pcsum - padded
    rank = jnp.arange(a, dtype=jnp.int32) - off[es]
    pslot = poff[es] + rank
    row_ids = jnp.zeros((np_,), jnp.int32).at[pslot].set(order // k)
    pos = jnp.zeros((a,), jnp.int32).at[order].set(pslot).reshape(n, k)
    na = pcsum[-1] // TILE
    te = jnp.searchsorted(
        pcsum, jnp.arange(nt, dtype=jnp.int32) * TILE, side="right"
    ).astype(jnp.int32)
    te = jnp.where(jnp.arange(nt) < na, te, te[na - 1])
    return row_ids, pos, te, na.reshape(1)


def _ffn_body(te_ref, na_ref, xs_ref, w1_ref, b1_ref, w2_ref, b2_ref,
              w3_ref, b3_ref, ys_ref):
    i = pl.program_id(0)

    @pl.when(i < na_ref[0])
    def _():
        x = xs_ref[...]
        dn = (((1,), (1,)), ((), ()))
        h1 = jax.lax.dot_general(
            x, w1_ref[0], dn, preferred_element_type=jnp.float32) + b1_ref[0]
        h3 = jax.lax.dot_general(
            x, w3_ref[0], dn, preferred_element_type=jnp.float32) + b3_ref[0]
        h = (h1 * jax.nn.sigmoid(h1)) * h3
        ys_ref[...] = jax.lax.dot_general(
            h, w2_ref[0], dn, preferred_element_type=jnp.float32) + b2_ref[0]


def _ffn(te, na, xs, W1, B1, W2, B2, W3, B3):
    np_, d = xs.shape
    e, inter, _ = W1.shape
    nt = np_ // TILE
    grid_spec = pltpu.PrefetchScalarGridSpec(
        num_scalar_prefetch=2,
        grid=(nt,),
        in_specs=[
            pl.BlockSpec((TILE, d), lambda i, te, na: (i, 0)),
            pl.BlockSpec((1, inter, d), lambda i, te, na: (te[i], 0, 0)),
            pl.BlockSpec((1, 1, inter), lambda i, te, na: (te[i], 0, 0)),
            pl.BlockSpec((1, d, inter), lambda i, te, na: (te[i], 0, 0)),
            pl.BlockSpec((1, 1, d), lambda i, te, na: (te[i], 0, 0)),
            pl.BlockSpec((1, inter, d), lambda i, te, na: (te[i], 0, 0)),
            pl.BlockSpec((1, 1, inter), lambda i, te, na: (te[i], 0, 0)),
        ],
        out_specs=pl.BlockSpec((TILE, d), lambda i, te, na: (i, 0)),
    )
    return pl.pallas_call(
        _ffn_body,
        grid_spec=grid_spec,
        out_shape=jax.ShapeDtypeStruct((np_, d), jnp.float32),
    )(te, na, xs, W1, B1.reshape(e, 1, inter), W2, B2.reshape(e, 1, d),
      W3, B3.reshape(e, 1, inter))


def kernel(x, Wg, bg, W1, B1, W2, B2, W3, B3):
    d = x.shape[-1]
    xf = x.reshape(-1, d)
    idx, w = _gating(xf, Wg, bg)
    row_ids, pos, te, na = _route(idx, Wg.shape[0])
    xs = xf[row_ids]
    ys = _ffn(te, na, xs, W1, B1, W2, B2, W3, B3)
    return w[:, :1] * ys[pos[:, 0]] + w[:, 1:] * ys[pos[:, 1]]


# speedup vs baseline: 5.4185x; 1.7629x over previous
import functools

import jax
import jax.numpy as jnp
from jax import lax
from jax.experimental import pallas as pl
from jax.experimental.pallas import tpu as pltpu
from jax.experimental.pallas import tpu_sc as plsc

TILE = 64
TOPK = 2
TPW = 64

_MESH = dict(core_axis_name="c", subcore_axis_name="s")


def _wid():
    return lax.axis_index("s") * 2 + lax.axis_index("c")


def _gating_body(xf_ref, wg_ref, bg_ref, w_ref, pos_ref, te_ref, na_ref):
    xf = xf_ref[...]
    logits = jax.lax.dot_general(
        xf, wg_ref[...], (((1,), (1,)), ((), ())),
        preferred_element_type=jnp.float32) + bg_ref[...]
    n, e = logits.shape
    nt = n * TOPK // TILE + e
    ids = jax.lax.broadcasted_iota(jnp.int32, (n, e), 1)
    m1 = jnp.max(logits, axis=1, keepdims=True)
    top1 = jnp.min(jnp.where(logits == m1, ids, e), axis=1)
    masked = jnp.where(ids == top1[:, None], -jnp.inf, logits)
    m2 = jnp.max(masked, axis=1, keepdims=True)
    top2 = jnp.min(jnp.where(masked == m2, ids, e), axis=1)
    w2 = jax.nn.sigmoid(m2[:, 0] - m1[:, 0])
    w_ref[...] = jnp.stack([1.0 - w2, w2], axis=1)
    h1 = (ids == top1[:, None]).astype(jnp.float32)
    h2 = (ids == top2[:, None]).astype(jnp.float32)
    hsum = h1 + h2
    counts = jnp.sum(hsum, axis=0, keepdims=True)
    padded = jnp.floor(counts / TILE - 1.0 / (2 * TILE)) * TILE + TILE
    ii = jax.lax.broadcasted_iota(jnp.int32, (e, e), 0)
    jj = jax.lax.broadcasted_iota(jnp.int32, (e, e), 1)
    tri = (ii <= jj).astype(jnp.float32)
    pcs = jax.lax.dot_general(padded, tri, (((1,), (0,)), ((), ())),
                              preferred_element_type=jnp.float32)
    po = pcs - padded
    ti = jax.lax.broadcasted_iota(jnp.int32, (n, n), 0)
    tj = jax.lax.broadcasted_iota(jnp.int32, (n, n), 1)
    ltri = (tj < ti).astype(jnp.float32)
    s = jax.lax.dot_general(ltri, hsum, (((1,), (0,)), ((), ())),
                            preferred_element_type=jnp.float32)
    slot1 = jnp.sum((s + po) * h1, axis=1)
    slot2 = jnp.sum((s + po) * h2, axis=1)
    pos_ref[...] = jnp.stack([slot1, slot2], axis=1).astype(jnp.int32)
    na = jnp.sum(padded, axis=1, keepdims=True) / TILE
    na_ref[...] = na.astype(jnp.int32)
    ci = jax.lax.broadcasted_iota(jnp.int32, (e, nt), 1)
    cmp = (pcs.reshape(e, 1) <= ci.astype(jnp.float32) * TILE).astype(
        jnp.float32)
    te_raw = jnp.sum(cmp, axis=0, keepdims=True)
    li = jax.lax.broadcasted_iota(jnp.int32, (1, nt), 1)
    el = jnp.sum(jnp.where(li == (na.astype(jnp.int32) - 1), te_raw, 0.0),
                 axis=1, keepdims=True)
    te_ref[...] = jnp.minimum(te_raw, el).astype(jnp.int32)


def _gating(xf, Wg, bg):
    n, d = xf.shape
    e = Wg.shape[0]
    nt = n * TOPK // TILE + e
    return pl.pallas_call(
        _gating_body,
        out_shape=(jax.ShapeDtypeStruct((n, TOPK), jnp.float32),
                   jax.ShapeDtypeStruct((n, TOPK), jnp.int32),
                   jax.ShapeDtypeStruct((1, nt), jnp.int32),
                   jax.ShapeDtypeStruct((1, 1), jnp.int32)),
    )(xf, Wg, bg.reshape(1, e))


def _dispatch_body(xf_hbm, p0_hbm, p1_hbm, xs_hbm, rows_v, p0_v, p1_v, sem):
    wid = _wid()
    tb = pl.multiple_of(wid * TPW, TPW)
    pltpu.sync_copy(xf_hbm.at[pl.ds(tb, TPW)], rows_v)
    pltpu.sync_copy(p0_hbm.at[pl.ds(tb, TPW)], p0_v)
    pltpu.sync_copy(p1_hbm.at[pl.ds(tb, TPW)], p1_v)
    c1 = pltpu.async_copy(rows_v, xs_hbm.at[p0_v], sem)
    c2 = pltpu.async_copy(rows_v, xs_hbm.at[p1_v], sem)
    c1.wait()
    c2.wait()


def _sc_dispatch(xf, p0, p1, np_):
    n, d = xf.shape
    f = pl.kernel(
        _dispatch_body,
        mesh=plsc.VectorSubcoreMesh(**_MESH),
        out_type=jax.ShapeDtypeStruct((np_, d), jnp.float32),
        scratch_types=[pltpu.VMEM((TPW, d), jnp.float32),
                       pltpu.VMEM((TPW,), jnp.int32),
                       pltpu.VMEM((TPW,), jnp.int32),
                       pltpu.SemaphoreType.DMA],
    )
    return f(xf, p0, p1)


def _ffn_body(te_ref, na_ref, xs_ref, w1_ref, b1_ref, w2_ref, b2_ref,
              w3_ref, b3_ref, ys_ref):
    i = pl.program_id(0)

    @pl.when(i < na_ref[0])
    def _():
        x = xs_ref[...]
        dn = (((1,), (1,)), ((), ()))
        h1 = jax.lax.dot_general(
            x, w1_ref[0], dn, preferred_element_type=jnp.float32) + b1_ref[0]
        h3 = jax.lax.dot_general(
            x, w3_ref[0], dn, preferred_element_type=jnp.float32) + b3_ref[0]
        h = (h1 * jax.nn.sigmoid(h1)) * h3
        ys_ref[...] = jax.lax.dot_general(
            h, w2_ref[0], dn, preferred_element_type=jnp.float32) + b2_ref[0]


def _ffn(te, na, xs, W1, B1, W2, B2, W3, B3):
    np_, d = xs.shape
    e, inter, _ = W1.shape
    nt = np_ // TILE
    grid_spec = pltpu.PrefetchScalarGridSpec(
        num_scalar_prefetch=2,
        grid=(nt,),
        in_specs=[
            pl.BlockSpec((TILE, d), lambda i, te, na: (i, 0)),
            pl.BlockSpec((1, inter, d), lambda i, te, na: (te[i], 0, 0)),
            pl.BlockSpec((1, 1, inter), lambda i, te, na: (te[i], 0, 0)),
            pl.BlockSpec((1, d, inter), lambda i, te, na: (te[i], 0, 0)),
            pl.BlockSpec((1, 1, d), lambda i, te, na: (te[i], 0, 0)),
            pl.BlockSpec((1, inter, d), lambda i, te, na: (te[i], 0, 0)),
            pl.BlockSpec((1, 1, inter), lambda i, te, na: (te[i], 0, 0)),
        ],
        out_specs=pl.BlockSpec((TILE, d), lambda i, te, na: (i, 0)),
    )
    return pl.pallas_call(
        _ffn_body,
        grid_spec=grid_spec,
        out_shape=jax.ShapeDtypeStruct((np_, d), jnp.float32),
    )(te, na, xs, W1, B1.reshape(e, 1, inter), W2, B2.reshape(e, 1, d),
      W3, B3.reshape(e, 1, inter))


def _combine_body(ys_hbm, p0_hbm, p1_hbm, w0_hbm, w1_hbm, y_hbm, p0_v, p1_v,
                  w0_v, w1_v, r0_v, r1_v, out_v, sem):
    d = ys_hbm.shape[1]
    wid = _wid()
    tb = pl.multiple_of(wid * TPW, TPW)
    pltpu.sync_copy(p0_hbm.at[pl.ds(tb, TPW)], p0_v)
    pltpu.sync_copy(p1_hbm.at[pl.ds(tb, TPW)], p1_v)
    pltpu.sync_copy(w0_hbm.at[pl.ds(tb, TPW)], w0_v.at[pl.ds(0, TPW)])
    pltpu.sync_copy(w1_hbm.at[pl.ds(tb, TPW)], w1_v.at[pl.ds(0, TPW)])
    for half in range(2):
        hb = half * (TPW // 2)
        c0 = pltpu.async_copy(
            ys_hbm.at[p0_v.at[pl.ds(hb, TPW // 2)]], r0_v, sem)
        c1 = pltpu.async_copy(
            ys_hbm.at[p1_v.at[pl.ds(hb, TPW // 2)]], r1_v, sem)
        c0.wait()
        c1.wait()

        def tok_body(t, _):
            w0s = w0_v[pl.ds(hb + t, 16)][0]
            w1s = w1_v[pl.ds(hb + t, 16)][0]
            for cc in range(d // 16):
                sl = pl.ds(cc * 16, 16)
                out_v[t, sl] = w0s * r0_v[t, sl] + w1s * r1_v[t, sl]
            return 0

        lax.fori_loop(0, TPW // 2, tok_body, 0)
        pltpu.sync_copy(
            out_v, y_hbm.at[pl.ds(pl.multiple_of(tb + hb, TPW // 2),
                                  TPW // 2)])


def _sc_combine(ys, p0, p1, w0, w1):
    np_, d = ys.shape
    n = p0.shape[0]
    f = pl.kernel(
        _combine_body,
        mesh=plsc.VectorSubcoreMesh(**_MESH),
        out_type=jax.ShapeDtypeStruct((n, d), jnp.float32),
        scratch_types=[pltpu.VMEM((TPW,), jnp.int32),
                       pltpu.VMEM((TPW,), jnp.int32),
                       pltpu.VMEM((TPW + 16,), jnp.float32),
                       pltpu.VMEM((TPW + 16,), jnp.float32),
                       pltpu.VMEM((TPW // 2, d), jnp.float32),
                       pltpu.VMEM((TPW // 2, d), jnp.float32),
                       pltpu.VMEM((TPW // 2, d), jnp.float32),
                       pltpu.SemaphoreType.DMA],
    )
    return f(ys, p0, p1, w0, w1)


def kernel(x, Wg, bg, W1, B1, W2, B2, W3, B3):
    d = x.shape[-1]
    e = Wg.shape[0]
    xf = x.reshape(-1, d)
    n = xf.shape[0]
    np_ = n * TOPK + e * TILE
    w, pos, te, na = _gating(xf, Wg, bg)
    p0 = pos[:, 0]
    p1 = pos[:, 1]
    xs = _sc_dispatch(xf, p0, p1, np_)
    ys = _ffn(te.reshape(-1), na.reshape(-1), xs, W1, B1, W2, B2, W3, B3)
    return _sc_combine(ys, p0, p1, w[:, 0], w[:, 1])


# speedup vs baseline: 6.4452x; 1.1895x over previous
import functools

import jax
import jax.numpy as jnp
from jax import lax
from jax.experimental import pallas as pl
from jax.experimental.pallas import tpu as pltpu
from jax.experimental.pallas import tpu_sc as plsc

TILE = 64
TOPK = 2
TPW = 64

_MESH = dict(core_axis_name="c", subcore_axis_name="s")


def _wid():
    return lax.axis_index("s") * 2 + lax.axis_index("c")


def _gating_body(xf_ref, wg_ref, bg_ref, w_ref, pos_ref, te_ref, na_ref,
                 first_ref, rank_ref, act_ref, nae_ref):
    xf = xf_ref[...]
    logits = jax.lax.dot_general(
        xf, wg_ref[...], (((1,), (1,)), ((), ())),
        preferred_element_type=jnp.float32) + bg_ref[...]
    n, e = logits.shape
    nt = n * TOPK // TILE + e
    ids = jax.lax.broadcasted_iota(jnp.int32, (n, e), 1)
    m1 = jnp.max(logits, axis=1, keepdims=True)
    top1 = jnp.min(jnp.where(logits == m1, ids, e), axis=1)
    masked = jnp.where(ids == top1[:, None], -jnp.inf, logits)
    m2 = jnp.max(masked, axis=1, keepdims=True)
    top2 = jnp.min(jnp.where(masked == m2, ids, e), axis=1)
    w2 = jax.nn.sigmoid(m2[:, 0] - m1[:, 0])
    w_ref[...] = jnp.stack([1.0 - w2, w2], axis=1)
    h1 = (ids == top1[:, None]).astype(jnp.float32)
    h2 = (ids == top2[:, None]).astype(jnp.float32)
    hsum = h1 + h2
    counts = jnp.sum(hsum, axis=0, keepdims=True)
    padded = jnp.floor(counts / TILE - 1.0 / (2 * TILE)) * TILE + TILE
    ii = jax.lax.broadcasted_iota(jnp.int32, (e, e), 0)
    jj = jax.lax.broadcasted_iota(jnp.int32, (e, e), 1)
    tri = (ii <= jj).astype(jnp.float32)
    pcs = jax.lax.dot_general(padded, tri, (((1,), (0,)), ((), ())),
                              preferred_element_type=jnp.float32)
    po = pcs - padded
    ti = jax.lax.broadcasted_iota(jnp.int32, (n, n), 0)
    tj = jax.lax.broadcasted_iota(jnp.int32, (n, n), 1)
    ltri = (tj < ti).astype(jnp.float32)
    s = jax.lax.dot_general(ltri, hsum, (((1,), (0,)), ((), ())),
                            preferred_element_type=jnp.float32)
    slot1 = jnp.sum((s + po) * h1, axis=1)
    slot2 = jnp.sum((s + po) * h2, axis=1)
    pos_ref[...] = jnp.stack([slot1, slot2], axis=1).astype(jnp.int32)
    na = jnp.sum(padded, axis=1, keepdims=True) / TILE
    na_ref[...] = na.astype(jnp.int32)
    ci = jax.lax.broadcasted_iota(jnp.int32, (e, nt), 1)
    cmp = (pcs.reshape(e, 1) <= ci.astype(jnp.float32) * TILE).astype(
        jnp.float32)
    te_raw = jnp.sum(cmp, axis=0, keepdims=True)
    li = jax.lax.broadcasted_iota(jnp.int32, (1, nt), 1)
    el = jnp.sum(jnp.where(li == (na.astype(jnp.int32) - 1), te_raw, 0.0),
                 axis=1, keepdims=True)
    te_ref[...] = jnp.minimum(te_raw, el).astype(jnp.int32)
    actm = (counts >= 0.5).astype(jnp.float32)
    tilev = ci.astype(jnp.float32) * TILE
    pov = (pcs - padded).reshape(e, 1)
    first_ref[...] = jnp.sum(
        actm.reshape(e, 1) * (pov == tilev).astype(jnp.float32),
        axis=0, keepdims=True).astype(jnp.int32)
    rank_ref[...] = jnp.sum(actm.reshape(e, 1) * cmp, axis=0,
                            keepdims=True).astype(jnp.int32)
    tris = (ii < jj).astype(jnp.float32)
    ranke = jax.lax.dot_general(actm, tris, (((1,), (0,)), ((), ())),
                                preferred_element_type=jnp.float32)
    ep = act_ref.shape[1]
    qi = jax.lax.broadcasted_iota(jnp.int32, (e, ep), 1).astype(jnp.float32)
    sel = (ranke.reshape(e, 1) == qi) * actm.reshape(e, 1)
    eidv = jax.lax.broadcasted_iota(jnp.int32, (e, ep), 0).astype(jnp.float32)
    act_ref[...] = jnp.sum(sel * eidv, axis=0, keepdims=True).astype(jnp.int32)
    nae_ref[...] = jnp.sum(actm, axis=1, keepdims=True).astype(jnp.int32)


def _gating(xf, Wg, bg):
    n, d = xf.shape
    e = Wg.shape[0]
    nt = n * TOPK // TILE + e
    return pl.pallas_call(
        _gating_body,
        out_shape=(jax.ShapeDtypeStruct((n, TOPK), jnp.float32),
                   jax.ShapeDtypeStruct((n, TOPK), jnp.int32),
                   jax.ShapeDtypeStruct((1, nt), jnp.int32),
                   jax.ShapeDtypeStruct((1, 1), jnp.int32),
                   jax.ShapeDtypeStruct((1, nt), jnp.int32),
                   jax.ShapeDtypeStruct((1, nt), jnp.int32),
                   jax.ShapeDtypeStruct((1, e + 4), jnp.int32),
                   jax.ShapeDtypeStruct((1, 1), jnp.int32)),
    )(xf, Wg, bg.reshape(1, e))


def _dispatch_body(xf_hbm, p0_hbm, p1_hbm, xs_hbm, rows_v, p0_v, p1_v, sem):
    wid = _wid()
    tb = pl.multiple_of(wid * TPW, TPW)
    pltpu.sync_copy(xf_hbm.at[pl.ds(tb, TPW)], rows_v)
    pltpu.sync_copy(p0_hbm.at[pl.ds(tb, TPW)], p0_v)
    pltpu.sync_copy(p1_hbm.at[pl.ds(tb, TPW)], p1_v)
    c1 = pltpu.async_copy(rows_v, xs_hbm.at[p0_v], sem)
    c2 = pltpu.async_copy(rows_v, xs_hbm.at[p1_v], sem)
    c1.wait()
    c2.wait()


def _sc_dispatch(xf, p0, p1, np_):
    n, d = xf.shape
    f = pl.kernel(
        _dispatch_body,
        mesh=plsc.VectorSubcoreMesh(**_MESH),
        out_type=jax.ShapeDtypeStruct((np_, d), jnp.float32),
        scratch_types=[pltpu.VMEM((TPW, d), jnp.float32),
                       pltpu.VMEM((TPW,), jnp.int32),
                       pltpu.VMEM((TPW,), jnp.int32),
                       pltpu.SemaphoreType.DMA],
    )
    return f(xf, p0, p1)


def _ffn_body(te_ref, na_ref, first_ref, rank_ref, act_ref, nae_ref,
              xs_ref, w1_ref, b1_ref, w2_ref, b2_ref, w3_ref, b3_ref,
              ys_ref, w1b, w2b, w3b, sems):
    i = pl.program_id(0)
    nae = nae_ref[0]

    def fetch(q, s):
        ex = act_ref[q]
        pltpu.make_async_copy(w1_ref.at[ex], w1b.at[s], sems.at[s]).start()
        pltpu.make_async_copy(w2_ref.at[ex], w2b.at[s], sems.at[s]).start()
        pltpu.make_async_copy(w3_ref.at[ex], w3b.at[s], sems.at[s]).start()

    def drain(q, s):
        ex = act_ref[q]
        pltpu.make_async_copy(w1_ref.at[ex], w1b.at[s], sems.at[s]).wait()
        pltpu.make_async_copy(w2_ref.at[ex], w2b.at[s], sems.at[s]).wait()
        pltpu.make_async_copy(w3_ref.at[ex], w3b.at[s], sems.at[s]).wait()

    @pl.when(i == 0)
    def _():
        fetch(0, 0)

    @pl.when((i == 0) & (nae > 1))
    def _():
        fetch(1, 1)

    q = rank_ref[i]
    is_first = (i < na_ref[0]) & (first_ref[i] == 1)

    @pl.when(is_first & (q + 2 < nae))
    def _():
        fetch(q + 2, lax.rem(q + 2, 3))

    @pl.when(is_first)
    def _():
        drain(q, lax.rem(q, 3))

    @pl.when(i < na_ref[0])
    def _():
        s = lax.rem(q, 3)
        x = xs_ref[...]
        dn = (((1,), (1,)), ((), ()))
        h1 = jax.lax.dot_general(
            x, w1b[s], dn, preferred_element_type=jnp.float32) + b1_ref[0]
        h3 = jax.lax.dot_general(
            x, w3b[s], dn, preferred_element_type=jnp.float32) + b3_ref[0]
        h = (h1 * jax.nn.sigmoid(h1)) * h3
        ys_ref[...] = jax.lax.dot_general(
            h, w2b[s], dn, preferred_element_type=jnp.float32) + b2_ref[0]


def _ffn(te, na, first, rank, act, nae, xs, W1, B1, W2, B2, W3, B3):
    np_, d = xs.shape
    e, inter, _ = W1.shape
    nt = np_ // TILE
    grid_spec = pltpu.PrefetchScalarGridSpec(
        num_scalar_prefetch=6,
        grid=(nt,),
        in_specs=[
            pl.BlockSpec((TILE, d), lambda i, *_: (i, 0)),
            pl.BlockSpec(memory_space=pl.ANY),
            pl.BlockSpec((1, 1, inter), lambda i, te, *_: (te[i], 0, 0)),
            pl.BlockSpec(memory_space=pl.ANY),
            pl.BlockSpec((1, 1, d), lambda i, te, *_: (te[i], 0, 0)),
            pl.BlockSpec(memory_space=pl.ANY),
            pl.BlockSpec((1, 1, inter), lambda i, te, *_: (te[i], 0, 0)),
        ],
        out_specs=pl.BlockSpec((TILE, d), lambda i, *_: (i, 0)),
        scratch_shapes=[
            pltpu.VMEM((3, inter, d), jnp.float32),
            pltpu.VMEM((3, d, inter), jnp.float32),
            pltpu.VMEM((3, inter, d), jnp.float32),
            pltpu.SemaphoreType.DMA((3,)),
        ],
    )
    return pl.pallas_call(
        _ffn_body,
        grid_spec=grid_spec,
        out_shape=jax.ShapeDtypeStruct((np_, d), jnp.float32),
    )(te, na, first, rank, act, nae, xs, W1, B1.reshape(e, 1, inter), W2,
      B2.reshape(e, 1, d), W3, B3.reshape(e, 1, inter))


def _combine_body(ys_hbm, p0_hbm, p1_hbm, w0_hbm, w1_hbm, y_hbm, p0_v, p1_v,
                  w0_v, w1_v, r0_v, r1_v, out_v, sem):
    d = ys_hbm.shape[1]
    wid = _wid()
    tb = pl.multiple_of(wid * TPW, TPW)
    pltpu.sync_copy(p0_hbm.at[pl.ds(tb, TPW)], p0_v)
    pltpu.sync_copy(p1_hbm.at[pl.ds(tb, TPW)], p1_v)
    pltpu.sync_copy(w0_hbm.at[pl.ds(tb, TPW)], w0_v.at[pl.ds(0, TPW)])
    pltpu.sync_copy(w1_hbm.at[pl.ds(tb, TPW)], w1_v.at[pl.ds(0, TPW)])
    for half in range(2):
        hb = half * (TPW // 2)
        c0 = pltpu.async_copy(
            ys_hbm.at[p0_v.at[pl.ds(hb, TPW // 2)]], r0_v, sem)
        c1 = pltpu.async_copy(
            ys_hbm.at[p1_v.at[pl.ds(hb, TPW // 2)]], r1_v, sem)
        c0.wait()
        c1.wait()

        def tok_body(t, _):
            w0s = w0_v[pl.ds(hb + t, 16)][0]
            w1s = w1_v[pl.ds(hb + t, 16)][0]
            for cc in range(d // 16):
                sl = pl.ds(cc * 16, 16)
                out_v[t, sl] = w0s * r0_v[t, sl] + w1s * r1_v[t, sl]
            return 0

        lax.fori_loop(0, TPW // 2, tok_body, 0)
        pltpu.sync_copy(
            out_v, y_hbm.at[pl.ds(pl.multiple_of(tb + hb, TPW // 2),
                                  TPW // 2)])


def _sc_combine(ys, p0, p1, w0, w1):
    np_, d = ys.shape
    n = p0.shape[0]
    f = pl.kernel(
        _combine_body,
        mesh=plsc.VectorSubcoreMesh(**_MESH),
        out_type=jax.ShapeDtypeStruct((n, d), jnp.float32),
        scratch_types=[pltpu.VMEM((TPW,), jnp.int32),
                       pltpu.VMEM((TPW,), jnp.int32),
                       pltpu.VMEM((TPW + 16,), jnp.float32),
                       pltpu.VMEM((TPW + 16,), jnp.float32),
                       pltpu.VMEM((TPW // 2, d), jnp.float32),
                       pltpu.VMEM((TPW // 2, d), jnp.float32),
                       pltpu.VMEM((TPW // 2, d), jnp.float32),
                       pltpu.SemaphoreType.DMA],
    )
    return f(ys, p0, p1, w0, w1)


def kernel(x, Wg, bg, W1, B1, W2, B2, W3, B3):
    d = x.shape[-1]
    e = Wg.shape[0]
    xf = x.reshape(-1, d)
    n = xf.shape[0]
    np_ = n * TOPK + e * TILE
    w, pos, te, na, first, rank, act, nae = _gating(xf, Wg, bg)
    p0 = pos[:, 0]
    p1 = pos[:, 1]
    xs = _sc_dispatch(xf, p0, p1, np_)
    ys = _ffn(te.reshape(-1), na.reshape(-1), first.reshape(-1),
              rank.reshape(-1), act.reshape(-1), nae.reshape(-1), xs,
              W1, B1, W2, B2, W3, B3)
    return _sc_combine(ys, p0, p1, w[:, 0], w[:, 1])


# speedup vs baseline: 6.4529x; 1.0012x over previous
import functools

import jax
import jax.numpy as jnp
from jax import lax
from jax.experimental import pallas as pl
from jax.experimental.pallas import tpu as pltpu
from jax.experimental.pallas import tpu_sc as plsc

TILE = 64
TOPK = 2
TPW = 64

_MESH = dict(core_axis_name="c", subcore_axis_name="s")


def _wid():
    return lax.axis_index("s") * 2 + lax.axis_index("c")


def _gating_body(xf_ref, wg_ref, bg_ref, w_ref, pos_ref, te_ref, na_ref,
                 first_ref, rank_ref, act_ref, nae_ref):
    xf = xf_ref[...]
    logits = jax.lax.dot_general(
        xf, wg_ref[...], (((1,), (1,)), ((), ())),
        preferred_element_type=jnp.float32) + bg_ref[...]
    n, e = logits.shape
    nt = n * TOPK // TILE + e
    ids = jax.lax.broadcasted_iota(jnp.int32, (n, e), 1)
    m1 = jnp.max(logits, axis=1, keepdims=True)
    top1 = jnp.min(jnp.where(logits == m1, ids, e), axis=1)
    masked = jnp.where(ids == top1[:, None], -jnp.inf, logits)
    m2 = jnp.max(masked, axis=1, keepdims=True)
    top2 = jnp.min(jnp.where(masked == m2, ids, e), axis=1)
    w2 = jax.nn.sigmoid(m2[:, 0] - m1[:, 0])
    w_ref[...] = jnp.stack([1.0 - w2, w2], axis=1)
    h1 = (ids == top1[:, None]).astype(jnp.float32)
    h2 = (ids == top2[:, None]).astype(jnp.float32)
    hsum = h1 + h2
    counts = jnp.sum(hsum, axis=0, keepdims=True)
    padded = jnp.floor(counts / TILE - 1.0 / (2 * TILE)) * TILE + TILE
    ii = jax.lax.broadcasted_iota(jnp.int32, (e, e), 0)
    jj = jax.lax.broadcasted_iota(jnp.int32, (e, e), 1)
    tri = (ii <= jj).astype(jnp.float32)
    pcs = jax.lax.dot_general(padded, tri, (((1,), (0,)), ((), ())),
                              preferred_element_type=jnp.float32)
    po = pcs - padded
    ti = jax.lax.broadcasted_iota(jnp.int32, (n, n), 0)
    tj = jax.lax.broadcasted_iota(jnp.int32, (n, n), 1)
    ltri = (tj < ti).astype(jnp.float32)
    s = jax.lax.dot_general(ltri, hsum, (((1,), (0,)), ((), ())),
                            preferred_element_type=jnp.float32)
    slot1 = jnp.sum((s + po) * h1, axis=1)
    slot2 = jnp.sum((s + po) * h2, axis=1)
    pos_ref[...] = jnp.stack([slot1, slot2], axis=1).astype(jnp.int32)
    na = jnp.sum(padded, axis=1, keepdims=True) / TILE
    na_ref[...] = na.astype(jnp.int32)
    ci = jax.lax.broadcasted_iota(jnp.int32, (e, nt), 1)
    cmp = (pcs.reshape(e, 1) <= ci.astype(jnp.float32) * TILE).astype(
        jnp.float32)
    te_raw = jnp.sum(cmp, axis=0, keepdims=True)
    li = jax.lax.broadcasted_iota(jnp.int32, (1, nt), 1)
    el = jnp.sum(jnp.where(li == (na.astype(jnp.int32) - 1), te_raw, 0.0),
                 axis=1, keepdims=True)
    te_ref[...] = jnp.minimum(te_raw, el).astype(jnp.int32)
    actm = (counts >= 0.5).astype(jnp.float32)
    tilev = ci.astype(jnp.float32) * TILE
    pov = (pcs - padded).reshape(e, 1)
    first_ref[...] = jnp.sum(
        actm.reshape(e, 1) * (pov == tilev).astype(jnp.float32),
        axis=0, keepdims=True).astype(jnp.int32)
    rank_ref[...] = jnp.sum(actm.reshape(e, 1) * cmp, axis=0,
                            keepdims=True).astype(jnp.int32)
    tris = (ii < jj).astype(jnp.float32)
    ranke = jax.lax.dot_general(actm, tris, (((1,), (0,)), ((), ())),
                                preferred_element_type=jnp.float32)
    ep = act_ref.shape[1]
    qi = jax.lax.broadcasted_iota(jnp.int32, (e, ep), 1).astype(jnp.float32)
    sel = (ranke.reshape(e, 1) == qi) * actm.reshape(e, 1)
    eidv = jax.lax.broadcasted_iota(jnp.int32, (e, ep), 0).astype(jnp.float32)
    act_ref[...] = jnp.sum(sel * eidv, axis=0, keepdims=True).astype(jnp.int32)
    nae_ref[...] = jnp.sum(actm, axis=1, keepdims=True).astype(jnp.int32)


def _gating(xf, Wg, bg):
    n, d = xf.shape
    e = Wg.shape[0]
    nt = n * TOPK // TILE + e
    return pl.pallas_call(
        _gating_body,
        out_shape=(jax.ShapeDtypeStruct((n, TOPK), jnp.float32),
                   jax.ShapeDtypeStruct((n, TOPK), jnp.int32),
                   jax.ShapeDtypeStruct((1, nt), jnp.int32),
                   jax.ShapeDtypeStruct((1, 1), jnp.int32),
                   jax.ShapeDtypeStruct((1, nt), jnp.int32),
                   jax.ShapeDtypeStruct((1, nt), jnp.int32),
                   jax.ShapeDtypeStruct((1, e + 4), jnp.int32),
                   jax.ShapeDtypeStruct((1, 1), jnp.int32)),
    )(xf, Wg, bg.reshape(1, e))


def _dispatch_body(xf_hbm, p0_hbm, p1_hbm, xs_hbm, rows_v, p0_v, p1_v, sem):
    wid = _wid()
    tb = pl.multiple_of(wid * TPW, TPW)
    pltpu.sync_copy(xf_hbm.at[pl.ds(tb, TPW)], rows_v)
    pltpu.sync_copy(p0_hbm.at[pl.ds(tb, TPW)], p0_v)
    pltpu.sync_copy(p1_hbm.at[pl.ds(tb, TPW)], p1_v)
    c1 = pltpu.async_copy(rows_v, xs_hbm.at[p0_v], sem)
    c2 = pltpu.async_copy(rows_v, xs_hbm.at[p1_v], sem)
    c1.wait()
    c2.wait()


def _sc_dispatch(xf, p0, p1, np_):
    n, d = xf.shape
    f = pl.kernel(
        _dispatch_body,
        mesh=plsc.VectorSubcoreMesh(**_MESH),
        out_type=jax.ShapeDtypeStruct((np_, d), jnp.float32),
        scratch_types=[pltpu.VMEM((TPW, d), jnp.float32),
                       pltpu.VMEM((TPW,), jnp.int32),
                       pltpu.VMEM((TPW,), jnp.int32),
                       pltpu.SemaphoreType.DMA],
    )
    return f(xf, p0, p1)


def _ffn_body(te_ref, na_ref, first_ref, rank_ref, act_ref, nae_ref,
              xs_ref, w1_ref, b1_ref, w2_ref, b2_ref, w3_ref, b3_ref,
              ys_ref, w1b, w2b, w3b, sems):
    i = pl.program_id(0)
    nae = nae_ref[0]

    def fetch(q, s):
        ex = act_ref[q]
        pltpu.make_async_copy(w1_ref.at[ex], w1b.at[s], sems.at[s]).start()
        pltpu.make_async_copy(w2_ref.at[ex], w2b.at[s], sems.at[s]).start()
        pltpu.make_async_copy(w3_ref.at[ex], w3b.at[s], sems.at[s]).start()

    def drain(q, s):
        ex = act_ref[q]
        pltpu.make_async_copy(w1_ref.at[ex], w1b.at[s], sems.at[s]).wait()
        pltpu.make_async_copy(w2_ref.at[ex], w2b.at[s], sems.at[s]).wait()
        pltpu.make_async_copy(w3_ref.at[ex], w3b.at[s], sems.at[s]).wait()

    @pl.when(i == 0)
    def _():
        fetch(0, 0)

    @pl.when((i == 0) & (nae > 1))
    def _():
        fetch(1, 1)

    q = rank_ref[i]
    is_first = (i < na_ref[0]) & (first_ref[i] == 1)

    @pl.when(is_first & (q + 2 < nae))
    def _():
        fetch(q + 2, lax.rem(q + 2, 3))

    @pl.when(is_first)
    def _():
        drain(q, lax.rem(q, 3))

    @pl.when(i < na_ref[0])
    def _():
        s = lax.rem(q, 3)
        x = xs_ref[...].astype(jnp.bfloat16)
        dn = (((1,), (1,)), ((), ()))
        h1 = jax.lax.dot_general(
            x, w1b[s].astype(jnp.bfloat16), dn,
            preferred_element_type=jnp.float32) + b1_ref[0]
        h3 = jax.lax.dot_general(
            x, w3b[s].astype(jnp.bfloat16), dn,
            preferred_element_type=jnp.float32) + b3_ref[0]
        h = ((h1 * jax.nn.sigmoid(h1)) * h3).astype(jnp.bfloat16)
        ys_ref[...] = jax.lax.dot_general(
            h, w2b[s].astype(jnp.bfloat16), dn,
            preferred_element_type=jnp.float32) + b2_ref[0]


def _ffn(te, na, first, rank, act, nae, xs, W1, B1, W2, B2, W3, B3):
    np_, d = xs.shape
    e, inter, _ = W1.shape
    nt = np_ // TILE
    grid_spec = pltpu.PrefetchScalarGridSpec(
        num_scalar_prefetch=6,
        grid=(nt,),
        in_specs=[
            pl.BlockSpec((TILE, d), lambda i, *_: (i, 0)),
            pl.BlockSpec(memory_space=pl.ANY),
            pl.BlockSpec((1, 1, inter), lambda i, te, *_: (te[i], 0, 0)),
            pl.BlockSpec(memory_space=pl.ANY),
            pl.BlockSpec((1, 1, d), lambda i, te, *_: (te[i], 0, 0)),
            pl.BlockSpec(memory_space=pl.ANY),
            pl.BlockSpec((1, 1, inter), lambda i, te, *_: (te[i], 0, 0)),
        ],
        out_specs=pl.BlockSpec((TILE, d), lambda i, *_: (i, 0)),
        scratch_shapes=[
            pltpu.VMEM((3, inter, d), jnp.float32),
            pltpu.VMEM((3, d, inter), jnp.float32),
            pltpu.VMEM((3, inter, d), jnp.float32),
            pltpu.SemaphoreType.DMA((3,)),
        ],
    )
    return pl.pallas_call(
        _ffn_body,
        grid_spec=grid_spec,
        out_shape=jax.ShapeDtypeStruct((np_, d), jnp.float32),
    )(te, na, first, rank, act, nae, xs, W1, B1.reshape(e, 1, inter), W2,
      B2.reshape(e, 1, d), W3, B3.reshape(e, 1, inter))


def _combine_body(ys_hbm, p0_hbm, p1_hbm, w0_hbm, w1_hbm, y_hbm, p0_v, p1_v,
                  w0_v, w1_v, r0_v, r1_v, out_v, sem):
    d = ys_hbm.shape[1]
    wid = _wid()
    tb = pl.multiple_of(wid * TPW, TPW)
    pltpu.sync_copy(p0_hbm.at[pl.ds(tb, TPW)], p0_v)
    pltpu.sync_copy(p1_hbm.at[pl.ds(tb, TPW)], p1_v)
    pltpu.sync_copy(w0_hbm.at[pl.ds(tb, TPW)], w0_v.at[pl.ds(0, TPW)])
    pltpu.sync_copy(w1_hbm.at[pl.ds(tb, TPW)], w1_v.at[pl.ds(0, TPW)])
    for half in range(2):
        hb = half * (TPW // 2)
        c0 = pltpu.async_copy(
            ys_hbm.at[p0_v.at[pl.ds(hb, TPW // 2)]], r0_v, sem)
        c1 = pltpu.async_copy(
            ys_hbm.at[p1_v.at[pl.ds(hb, TPW // 2)]], r1_v, sem)
        c0.wait()
        c1.wait()

        def tok_body(t, _):
            w0s = w0_v[pl.ds(hb + t, 16)][0]
            w1s = w1_v[pl.ds(hb + t, 16)][0]
            for cc in range(d // 16):
                sl = pl.ds(cc * 16, 16)
                out_v[t, sl] = w0s * r0_v[t, sl] + w1s * r1_v[t, sl]
            return 0

        lax.fori_loop(0, TPW // 2, tok_body, 0)
        pltpu.sync_copy(
            out_v, y_hbm.at[pl.ds(pl.multiple_of(tb + hb, TPW // 2),
                                  TPW // 2)])


def _sc_combine(ys, p0, p1, w0, w1):
    np_, d = ys.shape
    n = p0.shape[0]
    f = pl.kernel(
        _combine_body,
        mesh=plsc.VectorSubcoreMesh(**_MESH),
        out_type=jax.ShapeDtypeStruct((n, d), jnp.float32),
        scratch_types=[pltpu.VMEM((TPW,), jnp.int32),
                       pltpu.VMEM((TPW,), jnp.int32),
                       pltpu.VMEM((TPW + 16,), jnp.float32),
                       pltpu.VMEM((TPW + 16,), jnp.float32),
                       pltpu.VMEM((TPW // 2, d), jnp.float32),
                       pltpu.VMEM((TPW // 2, d), jnp.float32),
                       pltpu.VMEM((TPW // 2, d), jnp.float32),
                       pltpu.SemaphoreType.DMA],
    )
    return f(ys, p0, p1, w0, w1)


def kernel(x, Wg, bg, W1, B1, W2, B2, W3, B3):
    d = x.shape[-1]
    e = Wg.shape[0]
    xf = x.reshape(-1, d)
    n = xf.shape[0]
    np_ = n * TOPK + e * TILE
    w, pos, te, na, first, rank, act, nae = _gating(xf, Wg, bg)
    p0 = pos[:, 0]
    p1 = pos[:, 1]
    xs = _sc_dispatch(xf, p0, p1, np_)
    ys = _ffn(te.reshape(-1), na.reshape(-1), first.reshape(-1),
              rank.reshape(-1), act.reshape(-1), nae.reshape(-1), xs,
              W1, B1, W2, B2, W3, B3)
    return _sc_combine(ys, p0, p1, w[:, 0], w[:, 1])


# speedup vs baseline: 7.2303x; 1.1205x over previous
import functools

import jax
import jax.numpy as jnp
from jax import lax
from jax.experimental import pallas as pl
from jax.experimental.pallas import tpu as pltpu
from jax.experimental.pallas import tpu_sc as plsc

TILE = 128
TOPK = 2
TPW = 64

_MESH = dict(core_axis_name="c", subcore_axis_name="s")


def _wid():
    return lax.axis_index("s") * 2 + lax.axis_index("c")


def _gating_body(xf_ref, wg_ref, bg_ref, w_ref, pos_ref, te_ref, na_ref,
                 first_ref, rank_ref, act_ref, nae_ref):
    xf = xf_ref[...]
    logits = jax.lax.dot_general(
        xf, wg_ref[...], (((1,), (1,)), ((), ())),
        preferred_element_type=jnp.float32) + bg_ref[...]
    n, e = logits.shape
    nt = n * TOPK // TILE + e
    ids = jax.lax.broadcasted_iota(jnp.int32, (n, e), 1)
    m1 = jnp.max(logits, axis=1, keepdims=True)
    top1 = jnp.min(jnp.where(logits == m1, ids, e), axis=1)
    masked = jnp.where(ids == top1[:, None], -jnp.inf, logits)
    m2 = jnp.max(masked, axis=1, keepdims=True)
    top2 = jnp.min(jnp.where(masked == m2, ids, e), axis=1)
    w2 = jax.nn.sigmoid(m2[:, 0] - m1[:, 0])
    w_ref[...] = jnp.stack([1.0 - w2, w2], axis=1)
    h1 = (ids == top1[:, None]).astype(jnp.float32)
    h2 = (ids == top2[:, None]).astype(jnp.float32)
    hsum = h1 + h2
    counts = jnp.sum(hsum, axis=0, keepdims=True)
    padded = jnp.floor(counts / TILE - 1.0 / (2 * TILE)) * TILE + TILE
    ii = jax.lax.broadcasted_iota(jnp.int32, (e, e), 0)
    jj = jax.lax.broadcasted_iota(jnp.int32, (e, e), 1)
    tri = (ii <= jj).astype(jnp.float32)
    pcs = jax.lax.dot_general(padded, tri, (((1,), (0,)), ((), ())),
                              preferred_element_type=jnp.float32)
    po = pcs - padded
    ti = jax.lax.broadcasted_iota(jnp.int32, (n, n), 0)
    tj = jax.lax.broadcasted_iota(jnp.int32, (n, n), 1)
    ltri = (tj < ti).astype(jnp.float32)
    s = jax.lax.dot_general(ltri, hsum, (((1,), (0,)), ((), ())),
                            preferred_element_type=jnp.float32)
    slot1 = jnp.sum((s + po) * h1, axis=1)
    slot2 = jnp.sum((s + po) * h2, axis=1)
    pos_ref[...] = jnp.stack([slot1, slot2], axis=1).astype(jnp.int32)
    na = jnp.sum(padded, axis=1, keepdims=True) / TILE
    na_ref[...] = na.astype(jnp.int32)
    ci = jax.lax.broadcasted_iota(jnp.int32, (e, nt), 1)
    cmp = (pcs.reshape(e, 1) <= ci.astype(jnp.float32) * TILE).astype(
        jnp.float32)
    te_raw = jnp.sum(cmp, axis=0, keepdims=True)
    li = jax.lax.broadcasted_iota(jnp.int32, (1, nt), 1)
    el = jnp.sum(jnp.where(li == (na.astype(jnp.int32) - 1), te_raw, 0.0),
                 axis=1, keepdims=True)
    te_ref[...] = jnp.minimum(te_raw, el).astype(jnp.int32)
    actm = (counts >= 0.5).astype(jnp.float32)
    tilev = ci.astype(jnp.float32) * TILE
    pov = (pcs - padded).reshape(e, 1)
    first_ref[...] = jnp.sum(
        actm.reshape(e, 1) * (pov == tilev).astype(jnp.float32),
        axis=0, keepdims=True).astype(jnp.int32)
    rank_ref[...] = jnp.sum(actm.reshape(e, 1) * cmp, axis=0,
                            keepdims=True).astype(jnp.int32)
    tris = (ii < jj).astype(jnp.float32)
    ranke = jax.lax.dot_general(actm, tris, (((1,), (0,)), ((), ())),
                                preferred_element_type=jnp.float32)
    ep = act_ref.shape[1]
    qi = jax.lax.broadcasted_iota(jnp.int32, (e, ep), 1).astype(jnp.float32)
    sel = (ranke.reshape(e, 1) == qi) * actm.reshape(e, 1)
    eidv = jax.lax.broadcasted_iota(jnp.int32, (e, ep), 0).astype(jnp.float32)
    act_ref[...] = jnp.sum(sel * eidv, axis=0, keepdims=True).astype(jnp.int32)
    nae_ref[...] = jnp.sum(actm, axis=1, keepdims=True).astype(jnp.int32)


def _gating(xf, Wg, bg):
    n, d = xf.shape
    e = Wg.shape[0]
    nt = n * TOPK // TILE + e
    return pl.pallas_call(
        _gating_body,
        out_shape=(jax.ShapeDtypeStruct((n, TOPK), jnp.float32),
                   jax.ShapeDtypeStruct((n, TOPK), jnp.int32),
                   jax.ShapeDtypeStruct((1, nt), jnp.int32),
                   jax.ShapeDtypeStruct((1, 1), jnp.int32),
                   jax.ShapeDtypeStruct((1, nt), jnp.int32),
                   jax.ShapeDtypeStruct((1, nt), jnp.int32),
                   jax.ShapeDtypeStruct((1, e + 4), jnp.int32),
                   jax.ShapeDtypeStruct((1, 1), jnp.int32)),
    )(xf, Wg, bg.reshape(1, e))


def _dispatch_body(xf_hbm, p0_hbm, p1_hbm, xs_hbm, rows_v, p0_v, p1_v, sem):
    wid = _wid()
    tb = pl.multiple_of(wid * TPW, TPW)
    pltpu.sync_copy(xf_hbm.at[pl.ds(tb, TPW)], rows_v)
    pltpu.sync_copy(p0_hbm.at[pl.ds(tb, TPW)], p0_v)
    pltpu.sync_copy(p1_hbm.at[pl.ds(tb, TPW)], p1_v)
    c1 = pltpu.async_copy(rows_v, xs_hbm.at[p0_v], sem)
    c2 = pltpu.async_copy(rows_v, xs_hbm.at[p1_v], sem)
    c1.wait()
    c2.wait()


def _sc_dispatch(xf, p0, p1, np_):
    n, d = xf.shape
    f = pl.kernel(
        _dispatch_body,
        mesh=plsc.VectorSubcoreMesh(**_MESH),
        out_type=jax.ShapeDtypeStruct((np_, d), jnp.float32),
        scratch_types=[pltpu.VMEM((TPW, d), jnp.float32),
                       pltpu.VMEM((TPW,), jnp.int32),
                       pltpu.VMEM((TPW,), jnp.int32),
                       pltpu.SemaphoreType.DMA],
    )
    return f(xf, p0, p1)


RING = 4


def _ffn_body(te_ref, na_ref, first_ref, rank_ref, act_ref, nae_ref,
              xs_ref, w1_ref, b1_ref, w2_ref, b2_ref, w3_ref, b3_ref,
              ys_ref, w1b, w2b, w3b, w1c, w2c, w3c, sems):
    i = pl.program_id(0)
    nae = nae_ref[0]

    def fetch(q, s):
        ex = act_ref[q]
        pltpu.make_async_copy(w1_ref.at[ex], w1b.at[s], sems.at[s]).start()
        pltpu.make_async_copy(w2_ref.at[ex], w2b.at[s], sems.at[s]).start()
        pltpu.make_async_copy(w3_ref.at[ex], w3b.at[s], sems.at[s]).start()

    def drain(q, s):
        ex = act_ref[q]
        pltpu.make_async_copy(w1_ref.at[ex], w1b.at[s], sems.at[s]).wait()
        pltpu.make_async_copy(w2_ref.at[ex], w2b.at[s], sems.at[s]).wait()
        pltpu.make_async_copy(w3_ref.at[ex], w3b.at[s], sems.at[s]).wait()

    for r in range(RING - 1):
        @pl.when((i == 0) & (nae > r))
        def _(r=r):
            fetch(r, r)

    q = rank_ref[i]
    is_first = (i < na_ref[0]) & (first_ref[i] == 1)

    @pl.when(is_first & (q + RING - 1 < nae))
    def _():
        fetch(q + RING - 1, lax.rem(q + RING - 1, RING))

    @pl.when(is_first)
    def _():
        s = lax.rem(q, RING)
        drain(q, s)
        w1c[s] = w1b[s].astype(jnp.bfloat16)
        w2c[s] = w2b[s].astype(jnp.bfloat16)
        w3c[s] = w3b[s].astype(jnp.bfloat16)

    @pl.when(i < na_ref[0])
    def _():
        s = lax.rem(q, RING)
        x = xs_ref[...].astype(jnp.bfloat16)
        dn = (((1,), (1,)), ((), ()))
        h1 = jax.lax.dot_general(
            x, w1c[s], dn, preferred_element_type=jnp.float32) + b1_ref[0]
        h3 = jax.lax.dot_general(
            x, w3c[s], dn, preferred_element_type=jnp.float32) + b3_ref[0]
        h = ((h1 * jax.nn.sigmoid(h1)) * h3).astype(jnp.bfloat16)
        ys_ref[...] = jax.lax.dot_general(
            h, w2c[s], dn, preferred_element_type=jnp.float32) + b2_ref[0]


def _ffn(te, na, first, rank, act, nae, xs, W1, B1, W2, B2, W3, B3):
    np_, d = xs.shape
    e, inter, _ = W1.shape
    nt = np_ // TILE
    grid_spec = pltpu.PrefetchScalarGridSpec(
        num_scalar_prefetch=6,
        grid=(nt,),
        in_specs=[
            pl.BlockSpec((TILE, d), lambda i, *_: (i, 0)),
            pl.BlockSpec(memory_space=pl.ANY),
            pl.BlockSpec((1, 1, inter), lambda i, te, *_: (te[i], 0, 0)),
            pl.BlockSpec(memory_space=pl.ANY),
            pl.BlockSpec((1, 1, d), lambda i, te, *_: (te[i], 0, 0)),
            pl.BlockSpec(memory_space=pl.ANY),
            pl.BlockSpec((1, 1, inter), lambda i, te, *_: (te[i], 0, 0)),
        ],
        out_specs=pl.BlockSpec((TILE, d), lambda i, *_: (i, 0)),
        scratch_shapes=[
            pltpu.VMEM((RING, inter, d), jnp.float32),
            pltpu.VMEM((RING, d, inter), jnp.float32),
            pltpu.VMEM((RING, inter, d), jnp.float32),
            pltpu.VMEM((RING, inter, d), jnp.bfloat16),
            pltpu.VMEM((RING, d, inter), jnp.bfloat16),
            pltpu.VMEM((RING, inter, d), jnp.bfloat16),
            pltpu.SemaphoreType.DMA((RING,)),
        ],
    )
    return pl.pallas_call(
        _ffn_body,
        grid_spec=grid_spec,
        out_shape=jax.ShapeDtypeStruct((np_, d), jnp.float32),
    )(te, na, first, rank, act, nae, xs, W1, B1.reshape(e, 1, inter), W2,
      B2.reshape(e, 1, d), W3, B3.reshape(e, 1, inter))


def _combine_body(ys_hbm, p0_hbm, p1_hbm, w0_hbm, w1_hbm, y_hbm, p0_v, p1_v,
                  w0_v, w1_v, r0_v, r1_v, out_v, sem):
    d = ys_hbm.shape[1]
    wid = _wid()
    tb = pl.multiple_of(wid * TPW, TPW)
    pltpu.sync_copy(p0_hbm.at[pl.ds(tb, TPW)], p0_v)
    pltpu.sync_copy(p1_hbm.at[pl.ds(tb, TPW)], p1_v)
    pltpu.sync_copy(w0_hbm.at[pl.ds(tb, TPW)], w0_v.at[pl.ds(0, TPW)])
    pltpu.sync_copy(w1_hbm.at[pl.ds(tb, TPW)], w1_v.at[pl.ds(0, TPW)])
    for half in range(2):
        hb = half * (TPW // 2)
        c0 = pltpu.async_copy(
            ys_hbm.at[p0_v.at[pl.ds(hb, TPW // 2)]], r0_v, sem)
        c1 = pltpu.async_copy(
            ys_hbm.at[p1_v.at[pl.ds(hb, TPW // 2)]], r1_v, sem)
        c0.wait()
        c1.wait()

        def tok_body(t, _):
            w0s = w0_v[pl.ds(hb + t, 16)][0]
            w1s = w1_v[pl.ds(hb + t, 16)][0]
            for cc in range(d // 16):
                sl = pl.ds(cc * 16, 16)
                out_v[t, sl] = w0s * r0_v[t, sl] + w1s * r1_v[t, sl]
            return 0

        lax.fori_loop(0, TPW // 2, tok_body, 0)
        pltpu.sync_copy(
            out_v, y_hbm.at[pl.ds(pl.multiple_of(tb + hb, TPW // 2),
                                  TPW // 2)])


def _sc_combine(ys, p0, p1, w0, w1):
    np_, d = ys.shape
    n = p0.shape[0]
    f = pl.kernel(
        _combine_body,
        mesh=plsc.VectorSubcoreMesh(**_MESH),
        out_type=jax.ShapeDtypeStruct((n, d), jnp.float32),
        scratch_types=[pltpu.VMEM((TPW,), jnp.int32),
                       pltpu.VMEM((TPW,), jnp.int32),
                       pltpu.VMEM((TPW + 16,), jnp.float32),
                       pltpu.VMEM((TPW + 16,), jnp.float32),
                       pltpu.VMEM((TPW // 2, d), jnp.float32),
                       pltpu.VMEM((TPW // 2, d), jnp.float32),
                       pltpu.VMEM((TPW // 2, d), jnp.float32),
                       pltpu.SemaphoreType.DMA],
    )
    return f(ys, p0, p1, w0, w1)


def kernel(x, Wg, bg, W1, B1, W2, B2, W3, B3):
    d = x.shape[-1]
    e = Wg.shape[0]
    xf = x.reshape(-1, d)
    n = xf.shape[0]
    np_ = n * TOPK + e * TILE
    w, pos, te, na, first, rank, act, nae = _gating(xf, Wg, bg)
    p0 = pos[:, 0]
    p1 = pos[:, 1]
    xs = _sc_dispatch(xf, p0, p1, np_)
    ys = _ffn(te.reshape(-1), na.reshape(-1), first.reshape(-1),
              rank.reshape(-1), act.reshape(-1), nae.reshape(-1), xs,
              W1, B1, W2, B2, W3, B3)
    return _sc_combine(ys, p0, p1, w[:, 0], w[:, 1])


# speedup vs baseline: 7.2947x; 1.0089x over previous
import functools

import jax
import jax.numpy as jnp
from jax import lax
from jax.experimental import pallas as pl
from jax.experimental.pallas import tpu as pltpu
from jax.experimental.pallas import tpu_sc as plsc

TILE = 128
TOPK = 2
TPW = 64

_MESH = dict(core_axis_name="c", subcore_axis_name="s")


def _wid():
    return lax.axis_index("s") * 2 + lax.axis_index("c")


def _gating_body(xf_ref, wg_ref, bg_ref, w_ref, pos_ref, te_ref, na_ref,
                 first_ref, rank_ref, act_ref, nae_ref):
    xf = xf_ref[...]
    logits = jax.lax.dot_general(
        xf, wg_ref[...], (((1,), (1,)), ((), ())),
        preferred_element_type=jnp.float32) + bg_ref[...]
    n, e = logits.shape
    nt = n * TOPK // TILE + e
    ids = jax.lax.broadcasted_iota(jnp.int32, (n, e), 1)
    m1 = jnp.max(logits, axis=1, keepdims=True)
    top1 = jnp.min(jnp.where(logits == m1, ids, e), axis=1)
    masked = jnp.where(ids == top1[:, None], -jnp.inf, logits)
    m2 = jnp.max(masked, axis=1, keepdims=True)
    top2 = jnp.min(jnp.where(masked == m2, ids, e), axis=1)
    w2 = jax.nn.sigmoid(m2[:, 0] - m1[:, 0])
    w_ref[...] = jnp.stack([1.0 - w2, w2], axis=1)
    h1 = (ids == top1[:, None]).astype(jnp.float32)
    h2 = (ids == top2[:, None]).astype(jnp.float32)
    hsum = h1 + h2
    counts = jnp.sum(hsum, axis=0, keepdims=True)
    padded = jnp.floor(counts / TILE - 1.0 / (2 * TILE)) * TILE + TILE
    ii = jax.lax.broadcasted_iota(jnp.int32, (e, e), 0)
    jj = jax.lax.broadcasted_iota(jnp.int32, (e, e), 1)
    tri = (ii <= jj).astype(jnp.float32)
    pcs = jax.lax.dot_general(padded, tri, (((1,), (0,)), ((), ())),
                              preferred_element_type=jnp.float32)
    po = pcs - padded
    ti = jax.lax.broadcasted_iota(jnp.int32, (n, n), 0)
    tj = jax.lax.broadcasted_iota(jnp.int32, (n, n), 1)
    ltri = (tj < ti).astype(jnp.float32)
    s = jax.lax.dot_general(ltri, hsum, (((1,), (0,)), ((), ())),
                            preferred_element_type=jnp.float32)
    slot1 = jnp.sum((s + po) * h1, axis=1)
    slot2 = jnp.sum((s + po) * h2, axis=1)
    pos_ref[...] = jnp.stack([slot1, slot2], axis=1).astype(jnp.int32)
    na = jnp.sum(padded, axis=1, keepdims=True) / TILE
    na_ref[...] = na.astype(jnp.int32)
    ci = jax.lax.broadcasted_iota(jnp.int32, (e, nt), 1)
    cmp = (pcs.reshape(e, 1) <= ci.astype(jnp.float32) * TILE).astype(
        jnp.float32)
    te_raw = jnp.sum(cmp, axis=0, keepdims=True)
    li = jax.lax.broadcasted_iota(jnp.int32, (1, nt), 1)
    el = jnp.sum(jnp.where(li == (na.astype(jnp.int32) - 1), te_raw, 0.0),
                 axis=1, keepdims=True)
    te_ref[...] = jnp.minimum(te_raw, el).astype(jnp.int32)
    actm = (counts >= 0.5).astype(jnp.float32)
    tilev = ci.astype(jnp.float32) * TILE
    pov = (pcs - padded).reshape(e, 1)
    first_ref[...] = jnp.sum(
        actm.reshape(e, 1) * (pov == tilev).astype(jnp.float32),
        axis=0, keepdims=True).astype(jnp.int32)
    rank_ref[...] = jnp.sum(actm.reshape(e, 1) * cmp, axis=0,
                            keepdims=True).astype(jnp.int32)
    tris = (ii < jj).astype(jnp.float32)
    ranke = jax.lax.dot_general(actm, tris, (((1,), (0,)), ((), ())),
                                preferred_element_type=jnp.float32)
    ep = act_ref.shape[1]
    qi = jax.lax.broadcasted_iota(jnp.int32, (e, ep), 1).astype(jnp.float32)
    sel = (ranke.reshape(e, 1) == qi) * actm.reshape(e, 1)
    eidv = jax.lax.broadcasted_iota(jnp.int32, (e, ep), 0).astype(jnp.float32)
    act_ref[...] = jnp.sum(sel * eidv, axis=0, keepdims=True).astype(jnp.int32)
    nae_ref[...] = jnp.sum(actm, axis=1, keepdims=True).astype(jnp.int32)


def _gating(xf, Wg, bg):
    n, d = xf.shape
    e = Wg.shape[0]
    nt = n * TOPK // TILE + e
    return pl.pallas_call(
        _gating_body,
        out_shape=(jax.ShapeDtypeStruct((n, TOPK), jnp.float32),
                   jax.ShapeDtypeStruct((n, TOPK), jnp.int32),
                   jax.ShapeDtypeStruct((1, nt), jnp.int32),
                   jax.ShapeDtypeStruct((1, 1), jnp.int32),
                   jax.ShapeDtypeStruct((1, nt), jnp.int32),
                   jax.ShapeDtypeStruct((1, nt), jnp.int32),
                   jax.ShapeDtypeStruct((1, e + 4), jnp.int32),
                   jax.ShapeDtypeStruct((1, 1), jnp.int32)),
    )(xf, Wg, bg.reshape(1, e))


def _dispatch_body(xf_hbm, p0_hbm, p1_hbm, xs_hbm, rows_v, p0_v, p1_v, sem):
    wid = _wid()
    tb = pl.multiple_of(wid * TPW, TPW)
    pltpu.sync_copy(xf_hbm.at[pl.ds(tb, TPW)], rows_v)
    pltpu.sync_copy(p0_hbm.at[pl.ds(tb, TPW)], p0_v)
    pltpu.sync_copy(p1_hbm.at[pl.ds(tb, TPW)], p1_v)
    c1 = pltpu.async_copy(rows_v, xs_hbm.at[p0_v], sem)
    c2 = pltpu.async_copy(rows_v, xs_hbm.at[p1_v], sem)
    c1.wait()
    c2.wait()


def _sc_dispatch(xf, p0, p1, np_):
    n, d = xf.shape
    f = pl.kernel(
        _dispatch_body,
        mesh=plsc.VectorSubcoreMesh(**_MESH),
        out_type=jax.ShapeDtypeStruct((np_, d), jnp.float32),
        scratch_types=[pltpu.VMEM((TPW, d), jnp.float32),
                       pltpu.VMEM((TPW,), jnp.int32),
                       pltpu.VMEM((TPW,), jnp.int32),
                       pltpu.SemaphoreType.DMA],
    )
    return f(xf, p0, p1)


RING = 4


def _ffn_body(te_ref, na_ref, first_ref, rank_ref, act_ref, nae_ref,
              xs_ref, w1_ref, b1_ref, w2_ref, b2_ref, w3_ref, b3_ref,
              ys_ref, w1b, w2b, w3b, w1c, w2c, w3c, sems):
    i = pl.program_id(0)
    nae = nae_ref[0]

    def fetch(q, s):
        ex = act_ref[q]
        pltpu.make_async_copy(w1_ref.at[ex], w1b.at[s], sems.at[s]).start()
        pltpu.make_async_copy(w2_ref.at[ex], w2b.at[s], sems.at[s]).start()
        pltpu.make_async_copy(w3_ref.at[ex], w3b.at[s], sems.at[s]).start()

    def drain(q, s):
        ex = act_ref[q]
        pltpu.make_async_copy(w1_ref.at[ex], w1b.at[s], sems.at[s]).wait()
        pltpu.make_async_copy(w2_ref.at[ex], w2b.at[s], sems.at[s]).wait()
        pltpu.make_async_copy(w3_ref.at[ex], w3b.at[s], sems.at[s]).wait()

    for r in range(RING - 1):
        @pl.when((i == 0) & (nae > r))
        def _(r=r):
            fetch(r, r)

    q = rank_ref[i]
    is_first = (i < na_ref[0]) & (first_ref[i] == 1)

    @pl.when(is_first & (q + RING - 1 < nae))
    def _():
        fetch(q + RING - 1, lax.rem(q + RING - 1, RING))

    @pl.when(is_first)
    def _():
        s = lax.rem(q, RING)
        drain(q, s)
        w1c[s] = w1b[s].astype(jnp.bfloat16)
        w2c[s] = w2b[s].astype(jnp.bfloat16)
        w3c[s] = w3b[s].astype(jnp.bfloat16)

    @pl.when(i < na_ref[0])
    def _():
        s = lax.rem(q, RING)
        x = xs_ref[...].astype(jnp.bfloat16)
        dn = (((1,), (1,)), ((), ()))
        h1 = jax.lax.dot_general(
            x, w1c[s], dn, preferred_element_type=jnp.float32) + b1_ref[0]
        h3 = jax.lax.dot_general(
            x, w3c[s], dn, preferred_element_type=jnp.float32) + b3_ref[0]
        h = ((h1 * jax.nn.sigmoid(h1)) * h3).astype(jnp.bfloat16)
        ys_ref[...] = jax.lax.dot_general(
            h, w2c[s], dn, preferred_element_type=jnp.float32) + b2_ref[0]


def _ffn(te, na, first, rank, act, nae, xs, W1, B1, W2, B2, W3, B3):
    np_, d = xs.shape
    e, inter, _ = W1.shape
    nt = np_ // TILE
    grid_spec = pltpu.PrefetchScalarGridSpec(
        num_scalar_prefetch=6,
        grid=(nt,),
        in_specs=[
            pl.BlockSpec((TILE, d), lambda i, *_: (i, 0)),
            pl.BlockSpec(memory_space=pl.ANY),
            pl.BlockSpec((1, 1, inter), lambda i, te, *_: (te[i], 0, 0)),
            pl.BlockSpec(memory_space=pl.ANY),
            pl.BlockSpec((1, 1, d), lambda i, te, *_: (te[i], 0, 0)),
            pl.BlockSpec(memory_space=pl.ANY),
            pl.BlockSpec((1, 1, inter), lambda i, te, *_: (te[i], 0, 0)),
        ],
        out_specs=pl.BlockSpec((TILE, d), lambda i, *_: (i, 0)),
        scratch_shapes=[
            pltpu.VMEM((RING, inter, d), jnp.float32),
            pltpu.VMEM((RING, d, inter), jnp.float32),
            pltpu.VMEM((RING, inter, d), jnp.float32),
            pltpu.VMEM((RING, inter, d), jnp.bfloat16),
            pltpu.VMEM((RING, d, inter), jnp.bfloat16),
            pltpu.VMEM((RING, inter, d), jnp.bfloat16),
            pltpu.SemaphoreType.DMA((RING,)),
        ],
    )
    return pl.pallas_call(
        _ffn_body,
        grid_spec=grid_spec,
        out_shape=jax.ShapeDtypeStruct((np_, d), jnp.float32),
    )(te, na, first, rank, act, nae, xs, W1, B1.reshape(e, 1, inter), W2,
      B2.reshape(e, 1, d), W3, B3.reshape(e, 1, inter))


def _combine_body(ys_hbm, p0_hbm, p1_hbm, w0_hbm, w1_hbm, y_hbm, p0_v, p1_v,
                  w0_v, w1_v, r0a_v, r1a_v, r0b_v, r1b_v, sema, semb):
    d = ys_hbm.shape[1]
    wid = _wid()
    tb = pl.multiple_of(wid * TPW, TPW)
    pltpu.sync_copy(p0_hbm.at[pl.ds(tb, TPW)], p0_v)
    pltpu.sync_copy(p1_hbm.at[pl.ds(tb, TPW)], p1_v)
    pltpu.sync_copy(w0_hbm.at[pl.ds(tb, TPW)], w0_v.at[pl.ds(0, TPW)])
    pltpu.sync_copy(w1_hbm.at[pl.ds(tb, TPW)], w1_v.at[pl.ds(0, TPW)])
    hh = TPW // 2
    ca0 = pltpu.async_copy(ys_hbm.at[p0_v.at[pl.ds(0, hh)]], r0a_v, sema)
    ca1 = pltpu.async_copy(ys_hbm.at[p1_v.at[pl.ds(0, hh)]], r1a_v, sema)
    cb0 = pltpu.async_copy(ys_hbm.at[p0_v.at[pl.ds(hh, hh)]], r0b_v, semb)
    cb1 = pltpu.async_copy(ys_hbm.at[p1_v.at[pl.ds(hh, hh)]], r1b_v, semb)
    for half, r0_v, r1_v, c0, c1 in ((0, r0a_v, r1a_v, ca0, ca1),
                                     (1, r0b_v, r1b_v, cb0, cb1)):
        hb = half * hh
        c0.wait()
        c1.wait()

        def tok_body(t, _):
            w0s = w0_v[pl.ds(hb + t, 16)][0]
            w1s = w1_v[pl.ds(hb + t, 16)][0]
            for cc in range(d // 16):
                sl = pl.ds(cc * 16, 16)
                r0_v[t, sl] = w0s * r0_v[t, sl] + w1s * r1_v[t, sl]
            return 0

        lax.fori_loop(0, hh, tok_body, 0)
        pltpu.sync_copy(
            r0_v, y_hbm.at[pl.ds(pl.multiple_of(tb + hb, hh), hh)])


def _sc_combine(ys, p0, p1, w0, w1):
    np_, d = ys.shape
    n = p0.shape[0]
    f = pl.kernel(
        _combine_body,
        mesh=plsc.VectorSubcoreMesh(**_MESH),
        out_type=jax.ShapeDtypeStruct((n, d), jnp.float32),
        scratch_types=[pltpu.VMEM((TPW,), jnp.int32),
                       pltpu.VMEM((TPW,), jnp.int32),
                       pltpu.VMEM((TPW + 16,), jnp.float32),
                       pltpu.VMEM((TPW + 16,), jnp.float32),
                       pltpu.VMEM((TPW // 2, d), jnp.float32),
                       pltpu.VMEM((TPW // 2, d), jnp.float32),
                       pltpu.VMEM((TPW // 2, d), jnp.float32),
                       pltpu.VMEM((TPW // 2, d), jnp.float32),
                       pltpu.SemaphoreType.DMA,
                       pltpu.SemaphoreType.DMA],
    )
    return f(ys, p0, p1, w0, w1)


def kernel(x, Wg, bg, W1, B1, W2, B2, W3, B3):
    d = x.shape[-1]
    e = Wg.shape[0]
    xf = x.reshape(-1, d)
    n = xf.shape[0]
    np_ = n * TOPK + e * TILE
    w, pos, te, na, first, rank, act, nae = _gating(xf, Wg, bg)
    p0 = pos[:, 0]
    p1 = pos[:, 1]
    xs = _sc_dispatch(xf, p0, p1, np_)
    ys = _ffn(te.reshape(-1), na.reshape(-1), first.reshape(-1),
              rank.reshape(-1), act.reshape(-1), nae.reshape(-1), xs,
              W1, B1, W2, B2, W3, B3)
    return _sc_combine(ys, p0, p1, w[:, 0], w[:, 1])
